# baseline (speedup 1.0000x reference)
import functools

import jax
import jax.numpy as jnp
from jax import lax
from jax.experimental import pallas as pl
from jax.experimental.pallas import tpu as pltpu
from jax.experimental.pallas import tpu_sc as plsc

N_OUT = 65536
LANES = 16
NC, NS = 2, 16
NW = NC * NS
PER_W = N_OUT // NW
VECS = PER_W // LANES
CSR_PAD = N_OUT + 8


def _seen_body(csr_hbm, out_hbm, buf_v, out_v):
    wid = lax.axis_index("s") * NC + lax.axis_index("c")
    base = wid * PER_W
    pltpu.sync_copy(csr_hbm.at[pl.ds(base, PER_W + 8)], buf_v)

    def body(j, carry):
        lo = buf_v[pl.ds(j * LANES, LANES)]
        hi = buf_v[pl.ds(j * LANES + 1, LANES)]
        out_v[pl.ds(j * LANES, LANES)] = jnp.where(
            hi > lo,
            jnp.full((LANES,), 1, jnp.int32),
            jnp.zeros((LANES,), jnp.int32),
        )
        return carry

    lax.fori_loop(0, VECS, body, 0)
    pltpu.sync_copy(out_v, out_hbm.at[pl.ds(base, PER_W)])


_seen = functools.partial(
    pl.kernel,
    out_type=jax.ShapeDtypeStruct((N_OUT,), jnp.int32),
    mesh=plsc.VectorSubcoreMesh(core_axis_name="c", subcore_axis_name="s"),
    scratch_types=[
        pltpu.VMEM((PER_W + 8,), jnp.int32),
        pltpu.VMEM((PER_W,), jnp.int32),
    ],
)(_seen_body)


def kernel(x_3d, mod_x, csr_idx):
    csr = csr_idx.astype(jnp.int32)
    csr_pad = jnp.concatenate(
        [csr, jnp.zeros((CSR_PAD - csr.shape[0],), jnp.int32)]
    )
    seen = _seen(csr_pad)
    return (x_3d, mod_x, seen.astype(jnp.bool_))
